# baseline (speedup 1.0000x reference)
import functools

import jax
import jax.numpy as jnp
from jax import lax
from jax.experimental import pallas as pl
from jax.experimental.pallas import tpu as pltpu
from jax.experimental.pallas import tpu_sc as plsc

N, D, G, C = 10000, 128, 16, 7
NW = 32
CHUNK = N // NW
TAIL = N - NW * CHUNK
GA, GB = 128, 128
GC = CHUNK - GA - GB
IDS_PAD = 80 * 128

_MESH = plsc.VectorSubcoreMesh(core_axis_name="c", subcore_axis_name="s")


@functools.partial(
    pl.kernel,
    mesh=_MESH,
    out_type=jax.ShapeDtypeStruct((2, G, D), jnp.float32),
    scratch_types=[
        pltpu.VMEM((CHUNK, D), jnp.float32),
        pltpu.VMEM((GA,), jnp.int32),
        pltpu.VMEM((GB,), jnp.int32),
        pltpu.VMEM((GC,), jnp.int32),
        pltpu.VMEM((TAIL, D), jnp.float32),
        pltpu.VMEM((TAIL,), jnp.int32),
        pltpu.VMEM((G, D), jnp.float32),
        pltpu.VMEM_SHARED((G, D), jnp.float32),
    ],
)
def _sc_segment_sums(x_hbm, ids_hbm, sums_out,
                     x_v, ia_v, ib_v, ic_v, xt_v, it_v, zs_v, acc_sh):
    cid = lax.axis_index("c")
    sid = lax.axis_index("s")
    wid = sid * 2 + cid
    base = wid * CHUNK

    @pl.when(sid == 0)
    def _init():
        zvec = jnp.zeros((16,), jnp.float32)
        for r in range(G):
            for j in range(D // 16):
                zs_v[r, pl.ds(j * 16, 16)] = zvec
        pltpu.sync_copy(zs_v, acc_sh)

    pltpu.sync_copy(x_hbm.at[pl.ds(base, CHUNK)], x_v)
    pltpu.sync_copy(ids_hbm.at[pl.ds(base, GA)], ia_v)
    pltpu.sync_copy(ids_hbm.at[pl.ds(base + GA, GB)], ib_v)
    pltpu.sync_copy(ids_hbm.at[pl.ds(base + GA + GB, GC)], ic_v)

    @pl.when(wid == 0)
    def _stage_tail():
        pltpu.sync_copy(x_hbm.at[pl.ds(NW * CHUNK, TAIL)], xt_v)
        pltpu.sync_copy(ids_hbm.at[pl.ds(NW * CHUNK, TAIL)], it_v)

    plsc.subcore_barrier()

    pltpu.sync_copy(x_v.at[pl.ds(0, GA)], acc_sh.at[ia_v], add=True)
    pltpu.sync_copy(x_v.at[pl.ds(GA, GB)], acc_sh.at[ib_v], add=True)
    pltpu.sync_copy(x_v.at[pl.ds(GA + GB, GC)], acc_sh.at[ic_v], add=True)

    @pl.when(wid == 0)
    def _tail():
        pltpu.sync_copy(xt_v, acc_sh.at[it_v], add=True)

    plsc.subcore_barrier()

    @pl.when(sid == 0)
    def _flush():
        pltpu.sync_copy(acc_sh, zs_v)
        pltpu.sync_copy(zs_v, sums_out.at[cid])


def _tc_finish(sums_ref, ids_ref, Wc_ref, bc_ref, out_ref):
    s = sums_ref[0] + sums_ref[1]
    ids = ids_ref[...]
    rows = []
    for g in range(G):
        cg = jnp.sum((ids == g).astype(jnp.float32))
        rows.append(s[g:g + 1, :] / jnp.maximum(cg, 1.0))
    pooled = jnp.concatenate(rows, axis=0)
    out_ref[...] = jax.lax.dot(pooled, Wc_ref[...],
                               preferred_element_type=jnp.float32) \
        + bc_ref[...]


def kernel(x, edge_index, edge_attr, batch_size, W1, b1, W2, b2, Wc, bc):
    sums = _sc_segment_sums(x, batch_size)
    ids2d = jnp.pad(batch_size, (0, IDS_PAD - N),
                    constant_values=G).reshape(IDS_PAD // 128, 128)
    out = pl.pallas_call(
        _tc_finish,
        out_shape=jax.ShapeDtypeStruct((G, C), jnp.float32),
    )(sums, ids2d, Wc, bc.reshape(1, C))
    return out


# speedup vs baseline: 1.0839x; 1.0839x over previous
import functools

import jax
import jax.numpy as jnp
from jax import lax
from jax.experimental import pallas as pl
from jax.experimental.pallas import tpu as pltpu
from jax.experimental.pallas import tpu_sc as plsc

N, D, G, C = 10000, 128, 16, 7
NW = 32
CHUNK = N // NW
TAIL = N - NW * CHUNK
GA, GB = 128, 128
GC = CHUNK - GA - GB

_MESH = plsc.VectorSubcoreMesh(core_axis_name="c", subcore_axis_name="s")


@functools.partial(
    pl.kernel,
    mesh=_MESH,
    out_type=jax.ShapeDtypeStruct((2, G, D), jnp.float32),
    scratch_types=[
        pltpu.VMEM((CHUNK, D), jnp.float32),
        pltpu.VMEM((GA,), jnp.int32),
        pltpu.VMEM((GB,), jnp.int32),
        pltpu.VMEM((GC,), jnp.int32),
        pltpu.VMEM((TAIL, D), jnp.float32),
        pltpu.VMEM((TAIL,), jnp.int32),
        pltpu.VMEM((G, D), jnp.float32),
        pltpu.VMEM_SHARED((G, D), jnp.float32),
        pltpu.SemaphoreType.DMA,
        pltpu.SemaphoreType.DMA,
        pltpu.SemaphoreType.DMA,
        pltpu.SemaphoreType.DMA,
        pltpu.SemaphoreType.DMA,
        pltpu.SemaphoreType.DMA,
    ],
)
def _sc_segment_sums(x_hbm, ids_hbm, sums_out,
                     x_v, ia_v, ib_v, ic_v, xt_v, it_v, zs_v, acc_sh,
                     sa, sb, sc, si, ss, st):
    cid = lax.axis_index("c")
    sid = lax.axis_index("s")
    wid = sid * 2 + cid
    base = wid * CHUNK

    ca = pltpu.async_copy(x_hbm.at[pl.ds(base, GA)],
                          x_v.at[pl.ds(0, GA)], sa)
    cb = pltpu.async_copy(x_hbm.at[pl.ds(base + GA, GB)],
                          x_v.at[pl.ds(GA, GB)], sb)
    cc = pltpu.async_copy(x_hbm.at[pl.ds(base + GA + GB, GC)],
                          x_v.at[pl.ds(GA + GB, GC)], sc)
    ja = pltpu.async_copy(ids_hbm.at[pl.ds(base, GA)], ia_v, si)
    jb = pltpu.async_copy(ids_hbm.at[pl.ds(base + GA, GB)], ib_v, si)
    jc = pltpu.async_copy(ids_hbm.at[pl.ds(base + GA + GB, GC)], ic_v, si)

    @pl.when(wid == 0)
    def _stage_tail():
        pltpu.async_copy(x_hbm.at[pl.ds(NW * CHUNK, TAIL)], xt_v, st)
        pltpu.async_copy(ids_hbm.at[pl.ds(NW * CHUNK, TAIL)], it_v, st)

    @pl.when(sid == 0)
    def _init():
        zvec = jnp.zeros((16,), jnp.float32)
        for r in range(G):
            for j in range(D // 16):
                zs_v[r, pl.ds(j * 16, 16)] = zvec
        pltpu.sync_copy(zs_v, acc_sh)

    plsc.subcore_barrier()

    ja.wait()
    jb.wait()
    jc.wait()
    ca.wait()
    da = pltpu.async_copy(x_v.at[pl.ds(0, GA)], acc_sh.at[ia_v], ss,
                          add=True)
    cb.wait()
    db = pltpu.async_copy(x_v.at[pl.ds(GA, GB)], acc_sh.at[ib_v], ss,
                          add=True)
    cc.wait()
    dc = pltpu.async_copy(x_v.at[pl.ds(GA + GB, GC)], acc_sh.at[ic_v], ss,
                          add=True)

    @pl.when(wid == 0)
    def _tail():
        pltpu.make_async_copy(x_hbm.at[pl.ds(NW * CHUNK, TAIL)],
                              xt_v, st).wait()
        pltpu.make_async_copy(ids_hbm.at[pl.ds(NW * CHUNK, TAIL)],
                              it_v, st).wait()
        pltpu.sync_copy(xt_v, acc_sh.at[it_v], add=True)

    da.wait()
    db.wait()
    dc.wait()

    plsc.subcore_barrier()

    @pl.when(sid == 0)
    def _flush():
        pltpu.sync_copy(acc_sh, zs_v)
        pltpu.sync_copy(zs_v, sums_out.at[cid])


def _tc_finish(sums_ref, ids_ref, Wc_ref, bc_ref, out_ref):
    s = sums_ref[0] + sums_ref[1]
    ids = ids_ref[...]
    rows = []
    for g in range(G):
        cg = jnp.sum((ids == g).astype(jnp.float32))
        rows.append(s[g:g + 1, :] / jnp.maximum(cg, 1.0))
    pooled = jnp.concatenate(rows, axis=0)
    out_ref[...] = jax.lax.dot(pooled, Wc_ref[...],
                               preferred_element_type=jnp.float32) \
        + bc_ref[...]


def kernel(x, edge_index, edge_attr, batch_size, W1, b1, W2, b2, Wc, bc):
    sums = _sc_segment_sums(x, batch_size)
    ids3 = batch_size.reshape(10, 1, N // 10)
    out = pl.pallas_call(
        _tc_finish,
        out_shape=jax.ShapeDtypeStruct((G, C), jnp.float32),
    )(sums, ids3, Wc, bc.reshape(1, C))
    return out


# speedup vs baseline: 1.1068x; 1.0211x over previous
import functools

import jax
import jax.numpy as jnp
from jax import lax
from jax.experimental import pallas as pl
from jax.experimental.pallas import tpu as pltpu
from jax.experimental.pallas import tpu_sc as plsc

N, D, G, C = 10000, 128, 16, 7
NW = 32
CHUNK = 320
TAIL = N - 31 * CHUNK
GA, GB = 128, 128
GC = CHUNK - GA - GB

_MESH = plsc.VectorSubcoreMesh(core_axis_name="c", subcore_axis_name="s")


@functools.partial(
    pl.kernel,
    mesh=_MESH,
    out_type=jax.ShapeDtypeStruct((2, G, D), jnp.float32),
    scratch_types=[
        pltpu.VMEM((CHUNK, D), jnp.float32),
        pltpu.VMEM((GA,), jnp.int32),
        pltpu.VMEM((GB,), jnp.int32),
        pltpu.VMEM((GC,), jnp.int32),
        pltpu.VMEM((TAIL,), jnp.int32),
        pltpu.VMEM((G, D), jnp.float32),
        pltpu.VMEM_SHARED((G, D), jnp.float32),
        pltpu.SemaphoreType.DMA,
        pltpu.SemaphoreType.DMA,
        pltpu.SemaphoreType.DMA,
        pltpu.SemaphoreType.DMA,
        pltpu.SemaphoreType.DMA,
    ],
)
def _sc_segment_sums(x_hbm, ids_hbm, sums_out,
                     x_v, ia_v, ib_v, ic_v, it_v, zs_v, acc_sh,
                     sa, sb, sc, si, ss):
    cid = lax.axis_index("c")
    sid = lax.axis_index("s")
    wid = sid * 2 + cid
    base = wid * CHUNK
    is_tail = wid == NW - 1

    @pl.when(~is_tail)
    def _stage():
        pltpu.async_copy(ids_hbm.at[pl.ds(base, GA)], ia_v, si)
        pltpu.async_copy(ids_hbm.at[pl.ds(base + GA, GB)], ib_v, si)
        pltpu.async_copy(ids_hbm.at[pl.ds(base + GA + GB, GC)], ic_v, si)
        pltpu.async_copy(x_hbm.at[pl.ds(base, GA)],
                         x_v.at[pl.ds(0, GA)], sa)
        pltpu.async_copy(x_hbm.at[pl.ds(base + GA, GB)],
                         x_v.at[pl.ds(GA, GB)], sb)
        pltpu.async_copy(x_hbm.at[pl.ds(base + GA + GB, GC)],
                         x_v.at[pl.ds(GA + GB, GC)], sc)

    @pl.when(is_tail)
    def _stage_tail():
        pltpu.async_copy(ids_hbm.at[pl.ds(31 * CHUNK, TAIL)], it_v, si)
        pltpu.async_copy(x_hbm.at[pl.ds(31 * CHUNK, TAIL)],
                         x_v.at[pl.ds(0, TAIL)], sa)

    @pl.when(sid == 0)
    def _init():
        zvec = jnp.zeros((16,), jnp.float32)
        for r in range(G):
            for j in range(D // 16):
                zs_v[r, pl.ds(j * 16, 16)] = zvec
        pltpu.sync_copy(zs_v, acc_sh)

    plsc.subcore_barrier()

    @pl.when(~is_tail)
    def _main():
        pltpu.make_async_copy(ids_hbm.at[pl.ds(base, GA)], ia_v, si).wait()
        pltpu.make_async_copy(ids_hbm.at[pl.ds(base + GA, GB)],
                              ib_v, si).wait()
        pltpu.make_async_copy(ids_hbm.at[pl.ds(base + GA + GB, GC)],
                              ic_v, si).wait()
        pltpu.make_async_copy(x_hbm.at[pl.ds(base, GA)],
                              x_v.at[pl.ds(0, GA)], sa).wait()
        da = pltpu.async_copy(x_v.at[pl.ds(0, GA)], acc_sh.at[ia_v], ss,
                              add=True)
        pltpu.make_async_copy(x_hbm.at[pl.ds(base + GA, GB)],
                              x_v.at[pl.ds(GA, GB)], sb).wait()
        db = pltpu.async_copy(x_v.at[pl.ds(GA, GB)], acc_sh.at[ib_v], ss,
                              add=True)
        pltpu.make_async_copy(x_hbm.at[pl.ds(base + GA + GB, GC)],
                              x_v.at[pl.ds(GA + GB, GC)], sc).wait()
        dc = pltpu.async_copy(x_v.at[pl.ds(GA + GB, GC)], acc_sh.at[ic_v],
                              ss, add=True)
        da.wait()
        db.wait()
        dc.wait()

    @pl.when(is_tail)
    def _main_tail():
        pltpu.make_async_copy(ids_hbm.at[pl.ds(31 * CHUNK, TAIL)],
                              it_v, si).wait()
        pltpu.make_async_copy(x_hbm.at[pl.ds(31 * CHUNK, TAIL)],
                              x_v.at[pl.ds(0, TAIL)], sa).wait()
        pltpu.sync_copy(x_v.at[pl.ds(0, TAIL)], acc_sh.at[it_v], add=True)

    plsc.subcore_barrier()

    @pl.when(sid == 0)
    def _flush():
        pltpu.sync_copy(acc_sh, zs_v)
        pltpu.sync_copy(zs_v, sums_out.at[cid])


def _tc_finish(sums_ref, ids_ref, Wc_ref, bc_ref, out_ref):
    s = sums_ref[0] + sums_ref[1]
    cnt = jnp.zeros((G, 1), jnp.float32)
    gids = jax.lax.broadcasted_iota(jnp.int32, (G, N // 10), 0)
    for b in range(10):
        ids = ids_ref[b]
        cnt = cnt + jnp.sum((gids == ids).astype(jnp.float32),
                            axis=1, keepdims=True)
    recip = 1.0 / jnp.maximum(cnt, 1.0)
    pooled = s * recip
    out_ref[...] = jax.lax.dot(pooled, Wc_ref[...],
                               preferred_element_type=jnp.float32) \
        + bc_ref[...]


def kernel(x, edge_index, edge_attr, batch_size, W1, b1, W2, b2, Wc, bc):
    sums = _sc_segment_sums(x, batch_size)
    ids3 = batch_size.reshape(10, 1, N // 10)
    out = pl.pallas_call(
        _tc_finish,
        out_shape=jax.ShapeDtypeStruct((G, C), jnp.float32),
    )(sums, ids3, Wc, bc.reshape(1, C))
    return out


# speedup vs baseline: 1.1656x; 1.0531x over previous
import functools

import jax
import jax.numpy as jnp
from jax import lax
from jax.experimental import pallas as pl
from jax.experimental.pallas import tpu as pltpu
from jax.experimental.pallas import tpu_sc as plsc

N, D, G, C = 10000, 128, 16, 7
NW = 32
CHUNK = 320
TAIL = N - 31 * CHUNK
GA, GB = 128, 128
GC = CHUNK - GA - GB

_MESH = plsc.VectorSubcoreMesh(core_axis_name="c", subcore_axis_name="s")


@functools.partial(
    pl.kernel,
    mesh=_MESH,
    out_type=jax.ShapeDtypeStruct((2, G, D), jnp.float32),
    scratch_types=[
        pltpu.VMEM((CHUNK, D), jnp.float32),
        pltpu.VMEM((GA,), jnp.int32),
        pltpu.VMEM((GB,), jnp.int32),
        pltpu.VMEM((GC,), jnp.int32),
        pltpu.VMEM((TAIL,), jnp.int32),
        pltpu.VMEM((G, D), jnp.float32),
        pltpu.VMEM_SHARED((G, D), jnp.float32),
        pltpu.SemaphoreType.DMA,
        pltpu.SemaphoreType.DMA,
        pltpu.SemaphoreType.DMA,
        pltpu.SemaphoreType.DMA,
        pltpu.SemaphoreType.DMA,
    ],
)
def _sc_segment_sums(x_hbm, ids_hbm, sums_out,
                     x_v, ia_v, ib_v, ic_v, it_v, zs_v, acc_sh,
                     sa, sb, sc, si, ss):
    cid = lax.axis_index("c")
    sid = lax.axis_index("s")
    wid = sid * 2 + cid
    base = wid * CHUNK
    is_tail = wid == NW - 1

    @pl.when(~is_tail)
    def _stage():
        pltpu.async_copy(ids_hbm.at[pl.ds(base, GA)], ia_v, si)
        pltpu.async_copy(ids_hbm.at[pl.ds(base + GA, GB)], ib_v, si)
        pltpu.async_copy(ids_hbm.at[pl.ds(base + GA + GB, GC)], ic_v, si)
        pltpu.async_copy(x_hbm.at[pl.ds(base, GA)],
                         x_v.at[pl.ds(0, GA)], sa)
        pltpu.async_copy(x_hbm.at[pl.ds(base + GA, GB)],
                         x_v.at[pl.ds(GA, GB)], sb)
        pltpu.async_copy(x_hbm.at[pl.ds(base + GA + GB, GC)],
                         x_v.at[pl.ds(GA + GB, GC)], sc)

    @pl.when(is_tail)
    def _stage_tail():
        pltpu.async_copy(ids_hbm.at[pl.ds(31 * CHUNK, TAIL)], it_v, si)
        pltpu.async_copy(x_hbm.at[pl.ds(31 * CHUNK, TAIL)],
                         x_v.at[pl.ds(0, TAIL)], sa)

    @pl.when(sid == 0)
    def _init():
        zvec = jnp.zeros((16,), jnp.float32)
        for r in range(G):
            for j in range(D // 16):
                zs_v[r, pl.ds(j * 16, 16)] = zvec
        pltpu.sync_copy(zs_v, acc_sh)

    plsc.subcore_barrier()

    @pl.when(~is_tail)
    def _main():
        pltpu.make_async_copy(ids_hbm.at[pl.ds(base, GA)], ia_v, si).wait()
        pltpu.make_async_copy(ids_hbm.at[pl.ds(base + GA, GB)],
                              ib_v, si).wait()
        pltpu.make_async_copy(ids_hbm.at[pl.ds(base + GA + GB, GC)],
                              ic_v, si).wait()
        pltpu.make_async_copy(x_hbm.at[pl.ds(base, GA)],
                              x_v.at[pl.ds(0, GA)], sa).wait()
        da = pltpu.async_copy(x_v.at[pl.ds(0, GA)], acc_sh.at[ia_v], ss,
                              add=True)
        pltpu.make_async_copy(x_hbm.at[pl.ds(base + GA, GB)],
                              x_v.at[pl.ds(GA, GB)], sb).wait()
        db = pltpu.async_copy(x_v.at[pl.ds(GA, GB)], acc_sh.at[ib_v], ss,
                              add=True)
        pltpu.make_async_copy(x_hbm.at[pl.ds(base + GA + GB, GC)],
                              x_v.at[pl.ds(GA + GB, GC)], sc).wait()
        dc = pltpu.async_copy(x_v.at[pl.ds(GA + GB, GC)], acc_sh.at[ic_v],
                              ss, add=True)
        da.wait()
        db.wait()
        dc.wait()

    @pl.when(is_tail)
    def _main_tail():
        pltpu.make_async_copy(ids_hbm.at[pl.ds(31 * CHUNK, TAIL)],
                              it_v, si).wait()
        pltpu.make_async_copy(x_hbm.at[pl.ds(31 * CHUNK, TAIL)],
                              x_v.at[pl.ds(0, TAIL)], sa).wait()
        pltpu.sync_copy(x_v.at[pl.ds(0, TAIL)], acc_sh.at[it_v], add=True)

    plsc.subcore_barrier()

    @pl.when(sid == 0)
    def _flush():
        pltpu.sync_copy(acc_sh, zs_v)
        pltpu.sync_copy(zs_v, sums_out.at[cid])


def _tc_finish(sums_ref, ids_ref, WcT_ref, bcT_ref, outT_ref):
    s = sums_ref[0] + sums_ref[1]
    cnt = jnp.zeros((G, 1), jnp.float32)
    gids = jax.lax.broadcasted_iota(jnp.int32, (G, N // 10), 0)
    for b in range(10):
        ids = ids_ref[b]
        cnt = cnt + jnp.sum((gids == ids).astype(jnp.float32),
                            axis=1, keepdims=True)
    recip = 1.0 / jnp.maximum(cnt, 1.0)
    pooled = s * recip
    outT_ref[...] = jax.lax.dot_general(
        WcT_ref[...], pooled, (((1,), (1,)), ((), ())),
        preferred_element_type=jnp.float32) + bcT_ref[...]


def kernel(x, edge_index, edge_attr, batch_size, W1, b1, W2, b2, Wc, bc):
    sums = _sc_segment_sums(x, batch_size)
    ids3 = batch_size.reshape(10, 1, N // 10)
    outT = pl.pallas_call(
        _tc_finish,
        out_shape=jax.ShapeDtypeStruct((C, G), jnp.float32),
    )(sums, ids3, Wc.T, bc.reshape(C, 1))
    return outT.T


# speedup vs baseline: 1.1714x; 1.0050x over previous
import functools

import jax
import jax.numpy as jnp
from jax import lax
from jax.experimental import pallas as pl
from jax.experimental.pallas import tpu as pltpu
from jax.experimental.pallas import tpu_sc as plsc

N, D, G, C = 10000, 128, 16, 7
NW = 32
CHUNK = 320
TAIL = N - 31 * CHUNK
GRP = 64
NG = CHUNK // GRP
NGT = TAIL // GRP + 1
TG0, TG1 = 64, TAIL - 64

_MESH = plsc.VectorSubcoreMesh(core_axis_name="c", subcore_axis_name="s")


@functools.partial(
    pl.kernel,
    mesh=_MESH,
    out_type=jax.ShapeDtypeStruct((2, G, D), jnp.float32),
    scratch_types=(
        [pltpu.VMEM((CHUNK, D), jnp.float32)]
        + [pltpu.VMEM((GRP,), jnp.int32) for _ in range(NG)]
        + [pltpu.VMEM((TG1,), jnp.int32)]
        + [pltpu.VMEM((G, D), jnp.float32)]
        + [pltpu.VMEM_SHARED((G, D), jnp.float32)]
        + [pltpu.SemaphoreType.DMA for _ in range(NG)]
        + [pltpu.SemaphoreType.DMA,
           pltpu.SemaphoreType.DMA]
    ),
)
def _sc_segment_sums(x_hbm, ids_hbm, sums_out,
                     x_v, i0, i1, i2, i3, i4, it_v, zs_v, acc_sh,
                     s0, s1, s2, s3, s4, si, ss):
    idx_bufs = [i0, i1, i2, i3, i4]
    x_sems = [s0, s1, s2, s3, s4]
    cid = lax.axis_index("c")
    sid = lax.axis_index("s")
    wid = sid * 2 + cid
    base = wid * CHUNK
    is_tail = wid == NW - 1

    @pl.when(~is_tail)
    def _stage():
        for g in range(NG):
            pltpu.async_copy(ids_hbm.at[pl.ds(base + GRP * g, GRP)],
                             idx_bufs[g], si)
        for g in range(NG):
            pltpu.async_copy(x_hbm.at[pl.ds(base + GRP * g, GRP)],
                             x_v.at[pl.ds(GRP * g, GRP)], x_sems[g])

    @pl.when(is_tail)
    def _stage_tail():
        pltpu.async_copy(ids_hbm.at[pl.ds(31 * CHUNK, TG0)], i0, si)
        pltpu.async_copy(ids_hbm.at[pl.ds(31 * CHUNK + TG0, TG1)], it_v, si)
        pltpu.async_copy(x_hbm.at[pl.ds(31 * CHUNK, TG0)],
                         x_v.at[pl.ds(0, TG0)], s0)
        pltpu.async_copy(x_hbm.at[pl.ds(31 * CHUNK + TG0, TG1)],
                         x_v.at[pl.ds(TG0, TG1)], s1)

    @pl.when(sid == 0)
    def _init():
        zvec = jnp.zeros((16,), jnp.float32)
        for r in range(G):
            for j in range(D // 16):
                zs_v[r, pl.ds(j * 16, 16)] = zvec
        pltpu.sync_copy(zs_v, acc_sh)

    plsc.subcore_barrier()

    @pl.when(~is_tail)
    def _main():
        for g in range(NG):
            pltpu.make_async_copy(ids_hbm.at[pl.ds(base + GRP * g, GRP)],
                                  idx_bufs[g], si).wait()
        ds = []
        for g in range(NG):
            pltpu.make_async_copy(x_hbm.at[pl.ds(base + GRP * g, GRP)],
                                  x_v.at[pl.ds(GRP * g, GRP)],
                                  x_sems[g]).wait()
            ds.append(pltpu.async_copy(x_v.at[pl.ds(GRP * g, GRP)],
                                       acc_sh.at[idx_bufs[g]], ss,
                                       add=True))
        for d in ds:
            d.wait()

    @pl.when(is_tail)
    def _main_tail():
        pltpu.make_async_copy(ids_hbm.at[pl.ds(31 * CHUNK, TG0)],
                              i0, si).wait()
        pltpu.make_async_copy(ids_hbm.at[pl.ds(31 * CHUNK + TG0, TG1)],
                              it_v, si).wait()
        pltpu.make_async_copy(x_hbm.at[pl.ds(31 * CHUNK, TG0)],
                              x_v.at[pl.ds(0, TG0)], s0).wait()
        d0 = pltpu.async_copy(x_v.at[pl.ds(0, TG0)], acc_sh.at[i0], ss,
                              add=True)
        pltpu.make_async_copy(x_hbm.at[pl.ds(31 * CHUNK + TG0, TG1)],
                              x_v.at[pl.ds(TG0, TG1)], s1).wait()
        d1 = pltpu.async_copy(x_v.at[pl.ds(TG0, TG1)], acc_sh.at[it_v], ss,
                              add=True)
        d0.wait()
        d1.wait()

    plsc.subcore_barrier()

    @pl.when(sid == 0)
    def _flush():
        pltpu.sync_copy(acc_sh, zs_v)
        pltpu.sync_copy(zs_v, sums_out.at[cid])


def _tc_finish(sums_ref, ids_ref, WcT_ref, bcT_ref, outT_ref):
    s = sums_ref[0] + sums_ref[1]
    cnt = jnp.zeros((G, 1), jnp.float32)
    gids = jax.lax.broadcasted_iota(jnp.int32, (G, N // 10), 0)
    for b in range(10):
        ids = ids_ref[b]
        cnt = cnt + jnp.sum((gids == ids).astype(jnp.float32),
                            axis=1, keepdims=True)
    recip = 1.0 / jnp.maximum(cnt, 1.0)
    pooled = s * recip
    outT_ref[...] = jax.lax.dot_general(
        WcT_ref[...], pooled, (((1,), (1,)), ((), ())),
        preferred_element_type=jnp.float32) + bcT_ref[...]


def kernel(x, edge_index, edge_attr, batch_size, W1, b1, W2, b2, Wc, bc):
    sums = _sc_segment_sums(x, batch_size)
    ids3 = batch_size.reshape(10, 1, N // 10)
    outT = pl.pallas_call(
        _tc_finish,
        out_shape=jax.ShapeDtypeStruct((C, G), jnp.float32),
    )(sums, ids3, Wc.T, bc.reshape(C, 1))
    return outT.T
